# baseline (speedup 1.0000x reference)
import functools

import jax
import jax.numpy as jnp
from jax import lax
from jax.experimental import pallas as pl
from jax.experimental.pallas import tpu as pltpu
from jax.experimental.pallas import tpu_sc as plsc

F = 26
B = 4096
D = 32
NC = 2
NS = 16
NW = NC * NS
BPW = B // NW
ROWS_PW = F * BPW


def _sc_gather(gidx_hbm, w2f_hbm, w1f_hbm, x_out, e1_out,
               gidx_v, rows_v, e1_v, sem2, sem1):
    wid = lax.axis_index("s") * NC + lax.axis_index("c")
    pltpu.sync_copy(gidx_hbm.at[wid], gidx_v)
    cps = []
    for g in range(F):
        cps.append(pltpu.async_copy(
            w2f_hbm.at[gidx_v.at[g]], rows_v.at[pl.ds(g * BPW, BPW)], sem2))
        cps.append(pltpu.async_copy(
            w1f_hbm.at[gidx_v.at[g]], e1_v.at[g], sem1))
    for c in cps:
        c.wait()
    pltpu.sync_copy(rows_v, x_out.at[wid])
    pltpu.sync_copy(e1_v, e1_out.at[wid])


def _ln(x, g, b, eps=1e-5):
    mu = jnp.mean(x, axis=-1, keepdims=True)
    var = jnp.mean((x - mu) ** 2, axis=-1, keepdims=True)
    return (x - mu) / jnp.sqrt(var + eps) * g + b


def _tc_body(x_ref, e1_ref, W0_ref, b0_ref, g0_ref, be0_ref,
             W1_ref, b1_ref, g1_ref, be1_ref, w2r_ref, b2_ref, o_ref):
    x = x_ref[...]
    h = jnp.dot(x, W0_ref[...], preferred_element_type=jnp.float32) + b0_ref[...]
    h = jnp.maximum(_ln(h, g0_ref[...], be0_ref[...]), 0.0)
    h = jnp.dot(h, W1_ref[...], preferred_element_type=jnp.float32) + b1_ref[...]
    h = jnp.maximum(_ln(h, g1_ref[...], be1_ref[...]), 0.0)
    y_dnn = jnp.sum(h * w2r_ref[...], axis=-1, keepdims=True) + b2_ref[...]
    s = x[:, 0:D]
    for f in range(1, F):
        s = s + x[:, f * D:(f + 1) * D]
    y2 = 0.5 * (jnp.sum(s * s, axis=-1, keepdims=True)
                - jnp.sum(x * x, axis=-1, keepdims=True))
    y1 = jnp.sum(e1_ref[...], axis=-1, keepdims=True)
    o_ref[...] = jax.nn.sigmoid(y1 + y2 + y_dnn)


def kernel(indices, w1, w2, W0, b0, g0, be0, W1, b1, g1, be1, W2, b2):
    V = w2.shape[1]
    offs = jnp.arange(F, dtype=jnp.int32) * V
    gidx = (indices.T.astype(jnp.int32) + offs[None, :]).reshape(NW, F, BPW)

    w2f = w2.reshape(F * V, D)
    w1f = w1.reshape(F * V)

    sc = pl.kernel(
        _sc_gather,
        out_type=[
            jax.ShapeDtypeStruct((NW, ROWS_PW, D), jnp.float32),
            jax.ShapeDtypeStruct((NW, F, BPW), jnp.float32),
        ],
        mesh=plsc.VectorSubcoreMesh(core_axis_name="c", subcore_axis_name="s"),
        scratch_types=[
            pltpu.VMEM((F, BPW), jnp.int32),
            pltpu.VMEM((ROWS_PW, D), jnp.float32),
            pltpu.VMEM((F, BPW), jnp.float32),
            pltpu.SemaphoreType.DMA,
            pltpu.SemaphoreType.DMA,
        ],
        compiler_params=pltpu.CompilerParams(use_tc_tiling_on_sc=False),
    )
    x3, e1o = sc(gidx, w2f, w1f)
    x = x3.reshape(B, F * D)
    e1t = e1o.reshape(B, F)

    BT = 512
    grid = B // BT
    out2 = pl.pallas_call(
        _tc_body,
        grid=(grid,),
        in_specs=[
            pl.BlockSpec((BT, F * D), lambda i: (i, 0)),
            pl.BlockSpec((BT, F), lambda i: (i, 0)),
            pl.BlockSpec(W0.shape, lambda i: (0, 0)),
            pl.BlockSpec((1, b0.shape[0]), lambda i: (0, 0)),
            pl.BlockSpec((1, g0.shape[0]), lambda i: (0, 0)),
            pl.BlockSpec((1, be0.shape[0]), lambda i: (0, 0)),
            pl.BlockSpec(W1.shape, lambda i: (0, 0)),
            pl.BlockSpec((1, b1.shape[0]), lambda i: (0, 0)),
            pl.BlockSpec((1, g1.shape[0]), lambda i: (0, 0)),
            pl.BlockSpec((1, be1.shape[0]), lambda i: (0, 0)),
            pl.BlockSpec((1, W2.shape[0]), lambda i: (0, 0)),
            pl.BlockSpec((1, 1), lambda i: (0, 0)),
        ],
        out_specs=pl.BlockSpec((BT, 1), lambda i: (i, 0)),
        out_shape=jax.ShapeDtypeStruct((B, 1), jnp.float32),
    )(x, e1t, W0, b0.reshape(1, -1), g0.reshape(1, -1), be0.reshape(1, -1),
      W1, b1.reshape(1, -1), g1.reshape(1, -1), be1.reshape(1, -1),
      W2.reshape(1, -1), b2.reshape(1, 1))
    return out2[:, 0]


# speedup vs baseline: 7.1770x; 7.1770x over previous
import jax
import jax.numpy as jnp
from jax import lax
from jax.experimental import pallas as pl
from jax.experimental.pallas import tpu as pltpu
from jax.experimental.pallas import tpu_sc as plsc

F = 26
B = 4096
D = 32
NC = 2
NS = 16
NW = NC * NS
BPW = B // NW
CH = 16
NCH = BPW // CH


def _extract_scalar(vec, k):
    lane = lax.iota(jnp.int32, 16)
    return jnp.sum(jnp.where(lane == k, vec, jnp.zeros((16,), jnp.int32)))


def _sc_gather(gidx_hbm, w2_hbm, w1_hbm, x_out, y1_out,
               gidx_v, buf_a, buf_b, e1_a, e1_b, xstage, y1_v,
               sem2a, sem1a, sem2b, sem1b):
    wid = lax.axis_index("s") * NC + lax.axis_index("c")
    pltpu.sync_copy(gidx_hbm.at[wid], gidx_v)
    zeros16 = jnp.zeros((16,), jnp.float32)
    for m in range(BPW // 16):
        y1_v[pl.ds(m * 16, 16)] = zeros16

    bufs = (buf_a, buf_b)
    e1s = (e1_a, e1_b)
    sems2 = (sem2a, sem2b)
    sems1 = (sem1a, sem1b)

    def issue(g, c, p):
        buf, e1b = bufs[p], e1s[p]
        vec = gidx_v[g, pl.ds(c * CH, 16)]
        rbv = jnp.bitwise_and(vec, jnp.full((16,), -8, jnp.int32))
        for k in range(16):
            rb = pl.multiple_of(_extract_scalar(rbv, k), 8)
            pltpu.async_copy(w2_hbm.at[g, pl.ds(rb, 8)],
                             buf.at[pl.ds(k * 8, 8)], sems2[p])
            pltpu.async_copy(w1_hbm.at[g, pl.ds(rb, 8)],
                             e1b.at[pl.ds(k * 8, 8)], sems1[p])

    def drain_extract(g, c, p):
        buf, e1b = bufs[p], e1s[p]
        pltpu.make_async_copy(w2_hbm.at[g, pl.ds(0, CH * 8)],
                              buf, sems2[p]).wait()
        pltpu.make_async_copy(w1_hbm.at[g, pl.ds(0, CH * 8)],
                              e1b, sems1[p]).wait()
        vec = gidx_v[g, pl.ds(c * CH, 16)]
        rem = jnp.bitwise_and(vec, jnp.full((16,), 7, jnp.int32))
        ivec = lax.iota(jnp.int32, 16)
        subidx = ivec * 8 + rem
        for d in range(D):
            lanes = jnp.full((16,), d, jnp.int32)
            val = plsc.load_gather(buf, [subidx, lanes])
            xstage[d, pl.ds(c * CH, 16)] = val
        zlanes = jnp.zeros((16,), jnp.int32)
        e1v = plsc.load_gather(e1b, [subidx, zlanes])
        plsc.addupdate(y1_v.at[pl.ds(c * CH, 16)], e1v)

    def feature_body(g, carry):
        issue(g, 0, 0)
        for c in range(1, NCH):
            issue(g, c, c % 2)
            drain_extract(g, c - 1, (c - 1) % 2)
        drain_extract(g, NCH - 1, (NCH - 1) % 2)
        pltpu.sync_copy(xstage, x_out.at[wid, g])
        return carry

    lax.fori_loop(0, F, feature_body, 0)
    pltpu.sync_copy(y1_v, y1_out.at[wid])


def _ln(x, g, b, eps=1e-5):
    mu = jnp.mean(x, axis=-1, keepdims=True)
    var = jnp.mean((x - mu) ** 2, axis=-1, keepdims=True)
    return (x - mu) / jnp.sqrt(var + eps) * g + b


def _tc_body(x_ref, y1_ref, W0_ref, b0_ref, g0_ref, be0_ref,
             W1_ref, b1_ref, g1_ref, be1_ref, w2r_ref, b2_ref, o_ref):
    xcat = x_ref[0].reshape(F * D, BPW)
    h = lax.dot_general(xcat, W0_ref[...], (((0,), (0,)), ((), ())),
                        preferred_element_type=jnp.float32) + b0_ref[...]
    h = jnp.maximum(_ln(h, g0_ref[...], be0_ref[...]), 0.0)
    h = jnp.dot(h, W1_ref[...], preferred_element_type=jnp.float32) + b1_ref[...]
    h = jnp.maximum(_ln(h, g1_ref[...], be1_ref[...]), 0.0)
    y_dnn = jnp.sum(h * w2r_ref[...], axis=-1, keepdims=True) + b2_ref[...]
    s = x_ref[0, 0]
    for f in range(1, F):
        s = s + x_ref[0, f]
    y2row = 0.5 * (jnp.sum(s * s, axis=0, keepdims=True)
                   - jnp.sum(xcat * xcat, axis=0, keepdims=True))
    yrow = y1_ref[0] + y2row
    o_ref[...] = jax.nn.sigmoid(jnp.transpose(yrow) + y_dnn)


def kernel(indices, w1, w2, W0, b0, g0, be0, W1, b1, g1, be1, W2, b2):
    gidx = indices.astype(jnp.int32).reshape(F, NW, BPW).swapaxes(0, 1)

    sc = pl.kernel(
        _sc_gather,
        out_type=[
            jax.ShapeDtypeStruct((NW, F, D, BPW), jnp.float32),
            jax.ShapeDtypeStruct((NW, BPW), jnp.float32),
        ],
        mesh=plsc.VectorSubcoreMesh(core_axis_name="c", subcore_axis_name="s"),
        scratch_types=[
            pltpu.VMEM((F, BPW), jnp.int32),
            pltpu.VMEM((CH * 8, D), jnp.float32),
            pltpu.VMEM((CH * 8, D), jnp.float32),
            pltpu.VMEM((CH * 8, 1), jnp.float32),
            pltpu.VMEM((CH * 8, 1), jnp.float32),
            pltpu.VMEM((D, BPW), jnp.float32),
            pltpu.VMEM((BPW,), jnp.float32),
            pltpu.SemaphoreType.DMA,
            pltpu.SemaphoreType.DMA,
            pltpu.SemaphoreType.DMA,
            pltpu.SemaphoreType.DMA,
        ],
        compiler_params=pltpu.CompilerParams(needs_layout_passes=False),
    )
    x4, y1o = sc(gidx, w2, w1)

    grid = NW
    out2 = pl.pallas_call(
        _tc_body,
        grid=(grid,),
        in_specs=[
            pl.BlockSpec((1, F, D, BPW), lambda i: (i, 0, 0, 0)),
            pl.BlockSpec((1, 1, BPW), lambda i: (i, 0, 0)),
            pl.BlockSpec(W0.shape, lambda i: (0, 0)),
            pl.BlockSpec((1, b0.shape[0]), lambda i: (0, 0)),
            pl.BlockSpec((1, g0.shape[0]), lambda i: (0, 0)),
            pl.BlockSpec((1, be0.shape[0]), lambda i: (0, 0)),
            pl.BlockSpec(W1.shape, lambda i: (0, 0)),
            pl.BlockSpec((1, b1.shape[0]), lambda i: (0, 0)),
            pl.BlockSpec((1, g1.shape[0]), lambda i: (0, 0)),
            pl.BlockSpec((1, be1.shape[0]), lambda i: (0, 0)),
            pl.BlockSpec((1, W2.shape[0]), lambda i: (0, 0)),
            pl.BlockSpec((1, 1), lambda i: (0, 0)),
        ],
        out_specs=pl.BlockSpec((BPW, 1), lambda i: (i, 0)),
        out_shape=jax.ShapeDtypeStruct((B, 1), jnp.float32),
    )(x4, y1o.reshape(NW, 1, BPW), W0,
      b0.reshape(1, -1), g0.reshape(1, -1), be0.reshape(1, -1),
      W1, b1.reshape(1, -1), g1.reshape(1, -1), be1.reshape(1, -1),
      W2.reshape(1, -1), b2.reshape(1, 1))
    return out2[:, 0]
